# baseline (speedup 1.0000x reference)
import functools

import jax
import jax.numpy as jnp
from jax import lax
from jax.experimental import pallas as pl
from jax.experimental.pallas import tpu as pltpu
from jax.experimental.pallas import tpu_sc as plsc

BATCH = 16384
FACTOR = 32
NUM_CORES = 2
NUM_SUBCORES = 16
LANES = 16
NUM_WORKERS = NUM_CORES * NUM_SUBCORES
B_PER_W = BATCH // NUM_WORKERS
CHUNK = 128
CHUNKS_PER_W = B_PER_W // CHUNK


def _mf_body(user_ref, item_ref, avg_ref, eu_ref, ei_ref,
             predict_ref, u_ref, it_ref,
             idx_u, idx_it, rows_u, rows_it, pred_v, avg_v, sem):
    wid = lax.axis_index("s") * NUM_CORES + lax.axis_index("c")
    base = wid * B_PER_W
    crow = wid * CHUNKS_PER_W

    pltpu.sync_copy(user_ref.at[pl.ds(crow, CHUNKS_PER_W)], idx_u)
    pltpu.sync_copy(item_ref.at[pl.ds(crow, CHUNKS_PER_W)], idx_it)
    pltpu.sync_copy(avg_ref, avg_v)

    copies = []
    for g in range(CHUNKS_PER_W):
        copies.append(pltpu.async_copy(
            eu_ref.at[idx_u.at[g]], rows_u.at[pl.ds(g * CHUNK, CHUNK)], sem))
        copies.append(pltpu.async_copy(
            ei_ref.at[idx_it.at[g]], rows_it.at[pl.ds(g * CHUNK, CHUNK)], sem))
    for c in copies:
        c.wait()

    lane = lax.iota(jnp.int32, LANES)
    avg_vec = avg_v[...]

    def group_body(g, _):
        row_idx = g * LANES + lane

        def col_body(c, acc):
            col = jnp.full((LANES,), c, dtype=jnp.int32)
            uv = plsc.load_gather(rows_u, [row_idx, col])
            iv = plsc.load_gather(rows_it, [row_idx, col])
            return acc + uv * iv

        acc = lax.fori_loop(0, FACTOR, col_body,
                            jnp.zeros((LANES,), jnp.float32))
        pred_v[pl.ds(g * LANES, LANES)] = acc + avg_vec
        return 0

    lax.fori_loop(0, B_PER_W // LANES, group_body, 0)

    pltpu.sync_copy(rows_u, u_ref.at[pl.ds(base, B_PER_W)])
    pltpu.sync_copy(rows_it, it_ref.at[pl.ds(base, B_PER_W)])
    pltpu.sync_copy(pred_v, predict_ref.at[pl.ds(base, B_PER_W)])


@jax.jit
def _mf(user2d, item2d, avg16, embed_user, embed_item):
    mesh = plsc.VectorSubcoreMesh(core_axis_name="c", subcore_axis_name="s")
    f32 = jnp.float32
    kern = pl.kernel(
        _mf_body,
        out_type=(
            jax.ShapeDtypeStruct((BATCH,), f32),
            jax.ShapeDtypeStruct((BATCH, FACTOR), f32),
            jax.ShapeDtypeStruct((BATCH, FACTOR), f32),
        ),
        mesh=mesh,
        compiler_params=pltpu.CompilerParams(
            needs_layout_passes=False, use_tc_tiling_on_sc=False),
        scratch_types=[
            pltpu.VMEM((CHUNKS_PER_W, CHUNK), jnp.int32),
            pltpu.VMEM((CHUNKS_PER_W, CHUNK), jnp.int32),
            pltpu.VMEM((B_PER_W, FACTOR), f32),
            pltpu.VMEM((B_PER_W, FACTOR), f32),
            pltpu.VMEM((B_PER_W,), f32),
            pltpu.VMEM((LANES,), f32),
            pltpu.SemaphoreType.DMA,
        ],
    )
    return kern(user2d, item2d, avg16, embed_user, embed_item)


def kernel(user, item, average, embed_user, embed_item):
    user2d = user.astype(jnp.int32).reshape(BATCH // CHUNK, CHUNK)
    item2d = item.astype(jnp.int32).reshape(BATCH // CHUNK, CHUNK)
    avg16 = jnp.broadcast_to(average.astype(jnp.float32), (LANES,))
    predict, u, it = _mf(user2d, item2d, avg16, embed_user, embed_item)
    return predict, u, it


# speedup vs baseline: 2.6426x; 2.6426x over previous
import functools

import jax
import jax.numpy as jnp
from jax import lax
from jax.experimental import pallas as pl
from jax.experimental.pallas import tpu as pltpu
from jax.experimental.pallas import tpu_sc as plsc

BATCH = 16384
FACTOR = 32
VOCAB = 1000000
NUM_CORES = 2
NUM_SUBCORES = 16
LANES = 16
NUM_WORKERS = NUM_CORES * NUM_SUBCORES
B_PER_W = BATCH // NUM_WORKERS
GROUPS = B_PER_W // LANES
ROWS_PER_VROW = 128 // FACTOR


def _fetch_rows(tab_ref, idx_v, r0, buf, rows, sem):
    feat = lax.iota(jnp.int32, LANES)

    def group_body(g, _):
        row = r0 + g // 8
        col = (g % 8) * LANES
        rvec = idx_v[row, pl.ds(col, LANES)]
        for h in range(2):
            copies = []
            for jj in range(8):
                r = rvec[h * 8 + jj]
                tcb = pl.multiple_of((r // 128) * 128, 128)
                copies.append(pltpu.async_copy(
                    tab_ref.at[:, pl.ds(tcb, 128)], buf.at[jj], sem))
            for cp in copies:
                cp.wait()
            for jj in range(8):
                j = h * 8 + jj
                r = rvec[j]
                cv = jnp.full((LANES,), r % 128, dtype=jnp.int32)
                jv = jnp.full((LANES,), jj, dtype=jnp.int32)
                lo = plsc.load_gather(buf, [jv, feat, cv])
                hi = plsc.load_gather(buf, [jv, feat + LANES, cv])
                kflat = g * LANES + j
                rw = kflat // ROWS_PER_VROW
                cb = (kflat % ROWS_PER_VROW) * FACTOR
                rows[rw, pl.ds(cb, LANES)] = lo
                rows[rw, pl.ds(cb + LANES, LANES)] = hi
        return 0

    lax.fori_loop(0, GROUPS, group_body, 0)


def _mf_body(user_ref, item_ref, avg_ref, eu_ref, ei_ref,
             pred_ref, u_ref, it_ref,
             idx_u, idx_it, buf, rows_u, rows_it, pred2, avg_v, sem):
    wid = lax.axis_index("s") * NUM_CORES + lax.axis_index("c")
    crow8 = (wid // 2) * 8
    r0 = (wid % 2) * 4
    pltpu.sync_copy(user_ref.at[pl.ds(crow8, 8)], idx_u)
    pltpu.sync_copy(item_ref.at[pl.ds(crow8, 8)], idx_it)
    pltpu.sync_copy(avg_ref, avg_v)

    _fetch_rows(eu_ref, idx_u, r0, buf, rows_u, sem)
    _fetch_rows(ei_ref, idx_it, r0, buf, rows_it, sem)

    lane = lax.iota(jnp.int32, LANES)
    avg_vec = avg_v[...]

    def group_body(g, _):
        kvec = g * LANES + lane
        rvec = kvec // ROWS_PER_VROW
        cbase = (kvec % ROWS_PER_VROW) * FACTOR

        def col_body(c, acc):
            cc = cbase + c
            uv = plsc.load_gather(rows_u, [rvec, cc])
            iv = plsc.load_gather(rows_it, [rvec, cc])
            return acc + uv * iv

        acc = lax.fori_loop(0, FACTOR, col_body,
                            jnp.zeros((LANES,), jnp.float32))
        pred2[g // 2, pl.ds((g % 2) * LANES, LANES)] = acc + avg_vec
        return 0

    lax.fori_loop(0, GROUPS, group_body, 0)

    vbase = wid * (B_PER_W * FACTOR // 128)
    pltpu.sync_copy(rows_u, u_ref.at[pl.ds(vbase, 128)])
    pltpu.sync_copy(rows_it, it_ref.at[pl.ds(vbase, 128)])
    pltpu.sync_copy(pred2, pred_ref.at[pl.ds(wid * LANES, LANES)])


@jax.jit
def _mf(user2d, item2d, avg16, eu_t, ei_t):
    mesh = plsc.VectorSubcoreMesh(core_axis_name="c", subcore_axis_name="s")
    f32 = jnp.float32
    kern = pl.kernel(
        _mf_body,
        out_type=(
            jax.ShapeDtypeStruct((B_PER_W, FACTOR), f32),
            jax.ShapeDtypeStruct((BATCH * FACTOR // 128, 128), f32),
            jax.ShapeDtypeStruct((BATCH * FACTOR // 128, 128), f32),
        ),
        mesh=mesh,
        compiler_params=pltpu.CompilerParams(
            needs_layout_passes=False, use_tc_tiling_on_sc=True),
        scratch_types=[
            pltpu.VMEM((8, 128), jnp.int32),
            pltpu.VMEM((8, 128), jnp.int32),
            pltpu.VMEM((8, FACTOR, 128), f32),
            pltpu.VMEM((128, 128), f32),
            pltpu.VMEM((128, 128), f32),
            pltpu.VMEM((LANES, FACTOR), f32),
            pltpu.VMEM((LANES,), f32),
            pltpu.SemaphoreType.DMA,
        ],
    )
    return kern(user2d, item2d, avg16, eu_t, ei_t)


def kernel(user, item, average, embed_user, embed_item):
    user2d = user.astype(jnp.int32).reshape(BATCH // 128, 128)
    item2d = item.astype(jnp.int32).reshape(BATCH // 128, 128)
    avg16 = jnp.broadcast_to(average.astype(jnp.float32), (LANES,))
    pred2, u4, it4 = _mf(user2d, item2d, avg16, embed_user.T, embed_item.T)
    return (pred2.reshape(BATCH),
            u4.reshape(BATCH, FACTOR),
            it4.reshape(BATCH, FACTOR))


# speedup vs baseline: 3.1405x; 1.1884x over previous
import functools

import jax
import jax.numpy as jnp
from jax import lax
from jax.experimental import pallas as pl
from jax.experimental.pallas import tpu as pltpu
from jax.experimental.pallas import tpu_sc as plsc

BATCH = 16384
FACTOR = 32
VOCAB = 1000000
NUM_CORES = 2
NUM_SUBCORES = 16
LANES = 16
NUM_WORKERS = NUM_CORES * NUM_SUBCORES
B_PER_W = BATCH // NUM_WORKERS
GROUPS = B_PER_W // LANES
ROWS_PER_VROW = 128 // FACTOR


def _fetch_rows(tab_ref, idx_v, r0, buf0, buf1, rows, sem0, sem1):
    feat = lax.iota(jnp.int32, LANES)

    def load16(g):
        row = r0 + g // 8
        col = (g % 8) * LANES
        return idx_v[row, pl.ds(col, LANES)]

    def fire8(rvec, lanebase, buf, sem):
        for jj in range(8):
            r = rvec[lanebase + jj]
            tcb = pl.multiple_of((r // 128) * 128, 128)
            pltpu.async_copy(tab_ref.at[:, pl.ds(tcb, 128)], buf.at[jj], sem)

    def wait8(buf, sem):
        for jj in range(8):
            pltpu.make_async_copy(
                tab_ref.at[:, pl.ds(0, 128)], buf.at[jj], sem).wait()

    def extract8(rvec, lanebase, buf, g):
        for jj in range(8):
            j = lanebase + jj
            r = rvec[j]
            cv = jnp.full((LANES,), r % 128, dtype=jnp.int32)
            jv = jnp.full((LANES,), jj, dtype=jnp.int32)
            lo = plsc.load_gather(buf, [jv, feat, cv])
            hi = plsc.load_gather(buf, [jv, feat + LANES, cv])
            kflat = g * LANES + j
            rw = kflat // ROWS_PER_VROW
            cb = (kflat % ROWS_PER_VROW) * FACTOR
            rows[rw, pl.ds(cb, LANES)] = lo
            rows[rw, pl.ds(cb + LANES, LANES)] = hi

    fire8(load16(0), 0, buf0, sem0)

    def group_body(g, _):
        rvec = load16(g)
        fire8(rvec, 8, buf1, sem1)
        wait8(buf0, sem0)
        extract8(rvec, 0, buf0, g)

        @pl.when(g < GROUPS - 1)
        def _():
            fire8(load16(g + 1), 0, buf0, sem0)

        wait8(buf1, sem1)
        extract8(rvec, 8, buf1, g)
        return 0

    lax.fori_loop(0, GROUPS, group_body, 0)


def _mf_body(user_ref, item_ref, avg_ref, eu_ref, ei_ref,
             pred_ref, u_ref, it_ref,
             idx_u, idx_it, buf0, buf1, rows_u, rows_it, pred2, avg_v,
             sem0, sem1):
    wid = lax.axis_index("s") * NUM_CORES + lax.axis_index("c")
    crow8 = (wid // 2) * 8
    r0 = (wid % 2) * 4
    pltpu.sync_copy(user_ref.at[pl.ds(crow8, 8)], idx_u)
    pltpu.sync_copy(item_ref.at[pl.ds(crow8, 8)], idx_it)
    pltpu.sync_copy(avg_ref, avg_v)

    _fetch_rows(eu_ref, idx_u, r0, buf0, buf1, rows_u, sem0, sem1)
    _fetch_rows(ei_ref, idx_it, r0, buf0, buf1, rows_it, sem0, sem1)

    lane = lax.iota(jnp.int32, LANES)
    avg_vec = avg_v[...]

    def group_body(g, _):
        kvec = g * LANES + lane
        rvec = kvec // ROWS_PER_VROW
        cbase = (kvec % ROWS_PER_VROW) * FACTOR

        def col_body(c, acc):
            cc = cbase + c
            uv = plsc.load_gather(rows_u, [rvec, cc])
            iv = plsc.load_gather(rows_it, [rvec, cc])
            return acc + uv * iv

        acc = lax.fori_loop(0, FACTOR, col_body,
                            jnp.zeros((LANES,), jnp.float32))
        pred2[g // 2, pl.ds((g % 2) * LANES, LANES)] = acc + avg_vec
        return 0

    lax.fori_loop(0, GROUPS, group_body, 0)

    vbase = wid * (B_PER_W * FACTOR // 128)
    pltpu.sync_copy(rows_u, u_ref.at[pl.ds(vbase, 128)])
    pltpu.sync_copy(rows_it, it_ref.at[pl.ds(vbase, 128)])
    pltpu.sync_copy(pred2, pred_ref.at[pl.ds(wid * LANES, LANES)])


@jax.jit
def _mf(user2d, item2d, avg16, eu_t, ei_t):
    mesh = plsc.VectorSubcoreMesh(core_axis_name="c", subcore_axis_name="s")
    f32 = jnp.float32
    kern = pl.kernel(
        _mf_body,
        out_type=(
            jax.ShapeDtypeStruct((B_PER_W, FACTOR), f32),
            jax.ShapeDtypeStruct((BATCH * FACTOR // 128, 128), f32),
            jax.ShapeDtypeStruct((BATCH * FACTOR // 128, 128), f32),
        ),
        mesh=mesh,
        compiler_params=pltpu.CompilerParams(
            needs_layout_passes=False, use_tc_tiling_on_sc=True),
        scratch_types=[
            pltpu.VMEM((8, 128), jnp.int32),
            pltpu.VMEM((8, 128), jnp.int32),
            pltpu.VMEM((8, FACTOR, 128), f32),
            pltpu.VMEM((8, FACTOR, 128), f32),
            pltpu.VMEM((128, 128), f32),
            pltpu.VMEM((128, 128), f32),
            pltpu.VMEM((LANES, FACTOR), f32),
            pltpu.VMEM((LANES,), f32),
            pltpu.SemaphoreType.DMA,
            pltpu.SemaphoreType.DMA,
        ],
    )
    return kern(user2d, item2d, avg16, eu_t, ei_t)


def kernel(user, item, average, embed_user, embed_item):
    user2d = user.astype(jnp.int32).reshape(BATCH // 128, 128)
    item2d = item.astype(jnp.int32).reshape(BATCH // 128, 128)
    avg16 = jnp.broadcast_to(average.astype(jnp.float32), (LANES,))
    pred2, u4, it4 = _mf(user2d, item2d, avg16, embed_user.T, embed_item.T)
    return (pred2.reshape(BATCH),
            u4.reshape(BATCH, FACTOR),
            it4.reshape(BATCH, FACTOR))


# speedup vs baseline: 3.3921x; 1.0801x over previous
import functools

import jax
import jax.numpy as jnp
from jax import lax
from jax.experimental import pallas as pl
from jax.experimental.pallas import tpu as pltpu
from jax.experimental.pallas import tpu_sc as plsc

BATCH = 16384
FACTOR = 32
VOCAB = 1000000
NUM_CORES = 2
NUM_SUBCORES = 16
LANES = 16
NUM_WORKERS = NUM_CORES * NUM_SUBCORES
B_PER_W = BATCH // NUM_WORKERS
GROUPS = B_PER_W // LANES
ROWS_PER_VROW = 128 // FACTOR


def _fetch_rows(tab_ref, idx_v, r0, buf0, buf1, rows, sems):
    feat = lax.iota(jnp.int32, LANES)
    slots = [(buf0, 0), (buf0, 4), (buf1, 0), (buf1, 4)]

    def load16(g):
        row = r0 + g // 8
        col = (g % 8) * LANES
        return idx_v[row, pl.ds(col, LANES)]

    def fire4(rvec, q):
        buf, jb = slots[q]
        for jj in range(4):
            r = rvec[q * 4 + jj]
            tcb = pl.multiple_of((r // 128) * 128, 128)
            pltpu.async_copy(
                tab_ref.at[:, pl.ds(tcb, 128)], buf.at[jb + jj], sems[q])

    def wait4(q):
        buf, jb = slots[q]
        for jj in range(4):
            pltpu.make_async_copy(
                tab_ref.at[:, pl.ds(0, 128)], buf.at[jb + jj], sems[q]).wait()

    def extract4(rvec, q, g):
        buf, jb = slots[q]
        for jj in range(4):
            j = q * 4 + jj
            r = rvec[j]
            cv = jnp.full((LANES,), r % 128, dtype=jnp.int32)
            jv = jnp.full((LANES,), jb + jj, dtype=jnp.int32)
            lo = plsc.load_gather(buf, [jv, feat, cv])
            hi = plsc.load_gather(buf, [jv, feat + LANES, cv])
            kflat = g * LANES + j
            rw = kflat // ROWS_PER_VROW
            cb = (kflat % ROWS_PER_VROW) * FACTOR
            rows[rw, pl.ds(cb, LANES)] = lo
            rows[rw, pl.ds(cb + LANES, LANES)] = hi

    rvec0 = load16(0)
    for q in range(4):
        fire4(rvec0, q)

    def group_body(g, _):
        rvec = load16(g)
        rvnext = load16(jnp.minimum(g + 1, GROUPS - 1))
        for q in range(4):
            wait4(q)
            extract4(rvec, q, g)

            @pl.when(g < GROUPS - 1)
            def _():
                fire4(rvnext, q)

        return 0

    lax.fori_loop(0, GROUPS, group_body, 0)


def _mf_body(user_ref, item_ref, avg_ref, eu_ref, ei_ref,
             pred_ref, u_ref, it_ref,
             idx_u, idx_it, buf0, buf1, rows_u, rows_it, pred2, avg_v,
             sem0, sem1, sem2, sem3):
    wid = lax.axis_index("s") * NUM_CORES + lax.axis_index("c")
    crow8 = (wid // 2) * 8
    r0 = (wid % 2) * 4
    pltpu.sync_copy(user_ref.at[pl.ds(crow8, 8)], idx_u)
    pltpu.sync_copy(item_ref.at[pl.ds(crow8, 8)], idx_it)
    pltpu.sync_copy(avg_ref, avg_v)

    sems = (sem0, sem1, sem2, sem3)
    _fetch_rows(eu_ref, idx_u, r0, buf0, buf1, rows_u, sems)
    _fetch_rows(ei_ref, idx_it, r0, buf0, buf1, rows_it, sems)

    lane = lax.iota(jnp.int32, LANES)
    avg_vec = avg_v[...]

    def group_body(g, _):
        kvec = g * LANES + lane
        rvec = kvec // ROWS_PER_VROW
        cbase = (kvec % ROWS_PER_VROW) * FACTOR

        def col_body(c, acc):
            cc = cbase + c
            uv = plsc.load_gather(rows_u, [rvec, cc])
            iv = plsc.load_gather(rows_it, [rvec, cc])
            return acc + uv * iv

        acc = lax.fori_loop(0, FACTOR, col_body,
                            jnp.zeros((LANES,), jnp.float32))
        pred2[g // 2, pl.ds((g % 2) * LANES, LANES)] = acc + avg_vec
        return 0

    lax.fori_loop(0, GROUPS, group_body, 0)

    vbase = wid * (B_PER_W * FACTOR // 128)
    pltpu.sync_copy(rows_u, u_ref.at[pl.ds(vbase, 128)])
    pltpu.sync_copy(rows_it, it_ref.at[pl.ds(vbase, 128)])
    pltpu.sync_copy(pred2, pred_ref.at[pl.ds(wid * LANES, LANES)])


@jax.jit
def _mf(user2d, item2d, avg16, eu_t, ei_t):
    mesh = plsc.VectorSubcoreMesh(core_axis_name="c", subcore_axis_name="s")
    f32 = jnp.float32
    kern = pl.kernel(
        _mf_body,
        out_type=(
            jax.ShapeDtypeStruct((B_PER_W, FACTOR), f32),
            jax.ShapeDtypeStruct((BATCH * FACTOR // 128, 128), f32),
            jax.ShapeDtypeStruct((BATCH * FACTOR // 128, 128), f32),
        ),
        mesh=mesh,
        compiler_params=pltpu.CompilerParams(
            needs_layout_passes=False, use_tc_tiling_on_sc=True),
        scratch_types=[
            pltpu.VMEM((8, 128), jnp.int32),
            pltpu.VMEM((8, 128), jnp.int32),
            pltpu.VMEM((8, FACTOR, 128), f32),
            pltpu.VMEM((8, FACTOR, 128), f32),
            pltpu.VMEM((128, 128), f32),
            pltpu.VMEM((128, 128), f32),
            pltpu.VMEM((LANES, FACTOR), f32),
            pltpu.VMEM((LANES,), f32),
            pltpu.SemaphoreType.DMA,
            pltpu.SemaphoreType.DMA,
            pltpu.SemaphoreType.DMA,
            pltpu.SemaphoreType.DMA,
        ],
    )
    return kern(user2d, item2d, avg16, eu_t, ei_t)


def kernel(user, item, average, embed_user, embed_item):
    user2d = user.astype(jnp.int32).reshape(BATCH // 128, 128)
    item2d = item.astype(jnp.int32).reshape(BATCH // 128, 128)
    avg16 = jnp.broadcast_to(average.astype(jnp.float32), (LANES,))
    pred2, u4, it4 = _mf(user2d, item2d, avg16, embed_user.T, embed_item.T)
    return (pred2.reshape(BATCH),
            u4.reshape(BATCH, FACTOR),
            it4.reshape(BATCH, FACTOR))
